# baseline (speedup 1.0000x reference)
import jax
import jax.numpy as jnp
from jax.experimental import pallas as pl
from jax.experimental.pallas import tpu as pltpu

HIDDEN_DIM = 4096
NUM_EXPERTS = 64
PRED_HIDDEN = 256
TILE_M = 512
NBUF = 5

def _mlp_kernel(x_hbm, w1t_ref, b1_ref, w2t_ref, b2_ref, o_ref, buf_ref, sems):
    m = x_hbm.shape[0]
    num_tiles = m // TILE_M

    def copy_in(t):
        return pltpu.make_async_copy(
            x_hbm.at[pl.ds(t * TILE_M, TILE_M), :],
            buf_ref.at[t % NBUF],
            sems.at[t % NBUF],
        )

    for t in range(NBUF - 1):
        copy_in(t).start()

    w1t = w1t_ref[...]
    w2t = w2t_ref[...]
    b1 = b1_ref[...]
    b2 = b2_ref[...]

    def body(t, _):
        copy_in(t).wait()
        nxt = t + NBUF - 1

        @pl.when(nxt < num_tiles)
        def _():
            copy_in(nxt).start()

        xb = buf_ref[t % NBUF].astype(jnp.bfloat16)
        h = jnp.dot(xb, w1t, preferred_element_type=jnp.float32)
        h = jnp.maximum(h + b1, 0.0).astype(jnp.bfloat16)
        o_ref[pl.ds(t * TILE_M, TILE_M), :] = (
            jnp.dot(h, w2t, preferred_element_type=jnp.float32) + b2
        )
        return ()

    jax.lax.fori_loop(0, num_tiles, body, (), unroll=False)

def kernel(x, W1, b1, W2, b2, expert_bias):
    orig_shape = x.shape[:-1]
    x2 = x.reshape(-1, HIDDEN_DIM)
    m = x2.shape[0]
    w1t = W1.T.astype(jnp.bfloat16)
    w2t = W2.T.astype(jnp.bfloat16)
    b1r = b1.reshape(1, PRED_HIDDEN)
    b2r = (b2 + expert_bias).reshape(1, NUM_EXPERTS)
    out = pl.pallas_call(
        _mlp_kernel,
        in_specs=[
            pl.BlockSpec(memory_space=pl.ANY),
            pl.BlockSpec(memory_space=pltpu.VMEM),
            pl.BlockSpec(memory_space=pltpu.VMEM),
            pl.BlockSpec(memory_space=pltpu.VMEM),
            pl.BlockSpec(memory_space=pltpu.VMEM),
        ],
        out_specs=pl.BlockSpec(memory_space=pltpu.VMEM),
        out_shape=jax.ShapeDtypeStruct((m, NUM_EXPERTS), jnp.float32),
        scratch_shapes=[
            pltpu.VMEM((NBUF, TILE_M, HIDDEN_DIM), jnp.float32),
            pltpu.SemaphoreType.DMA((NBUF,)),
        ],
    )(x2, w1t, b1r, w2t, b2r)
    return out.reshape(*orig_shape, NUM_EXPERTS)


# speedup vs baseline: 1.0341x; 1.0341x over previous
import jax
import jax.numpy as jnp
from jax.experimental import pallas as pl
from jax.experimental.pallas import tpu as pltpu

HIDDEN_DIM = 4096
NUM_EXPERTS = 64
PRED_HIDDEN = 256
TILE_M = 1024


def _mlp_kernel(x_ref, w1t_ref, b1_ref, w2t_ref, b2_ref, o_ref):
    xb = x_ref[...].astype(jnp.bfloat16)
    h = jnp.dot(xb, w1t_ref[...], preferred_element_type=jnp.float32)
    h = jnp.maximum(h + b1_ref[...], 0.0).astype(jnp.bfloat16)
    o_ref[...] = (
        jnp.dot(h, w2t_ref[...], preferred_element_type=jnp.float32) + b2_ref[...]
    )


def kernel(x, W1, b1, W2, b2, expert_bias):
    orig_shape = x.shape[:-1]
    x2 = x.reshape(-1, HIDDEN_DIM)
    m = x2.shape[0]
    w1t = W1.T.astype(jnp.bfloat16)
    w2t = W2.T.astype(jnp.bfloat16)
    b1r = b1.reshape(1, PRED_HIDDEN)
    b2r = (b2 + expert_bias).reshape(1, NUM_EXPERTS)

    out = pl.pallas_call(
        _mlp_kernel,
        grid=(m // TILE_M,),
        in_specs=[
            pl.BlockSpec((TILE_M, HIDDEN_DIM), lambda i: (i, 0)),
            pl.BlockSpec((HIDDEN_DIM, PRED_HIDDEN), lambda i: (0, 0)),
            pl.BlockSpec((1, PRED_HIDDEN), lambda i: (0, 0)),
            pl.BlockSpec((PRED_HIDDEN, NUM_EXPERTS), lambda i: (0, 0)),
            pl.BlockSpec((1, NUM_EXPERTS), lambda i: (0, 0)),
        ],
        out_specs=pl.BlockSpec((TILE_M, NUM_EXPERTS), lambda i: (i, 0)),
        out_shape=jax.ShapeDtypeStruct((m, NUM_EXPERTS), jnp.float32),
        compiler_params=pltpu.CompilerParams(
            dimension_semantics=("parallel",),
        ),
    )(x2, w1t, b1r, w2t, b2r)
    return out.reshape(*orig_shape, NUM_EXPERTS)
